# baseline (speedup 1.0000x reference)
import functools

import jax
import jax.numpy as jnp
from jax import lax
from jax.experimental import pallas as pl
from jax.experimental.pallas import tpu as pltpu
from jax.experimental.pallas import tpu_sc as plsc

D = 64
M = 128

_NC = 2
_NS = 16
_NW = _NC * _NS

_CHUNK = 512
_TCBLK = 4096


def _gather_body(table_hbm, ids_hbm, out_hbm, idx_v, rows_v, sem):
    wid = lax.axis_index("s") * _NC + lax.axis_index("c")
    n = ids_hbm.shape[0]
    b_per_w = n // _NW
    n_chunks = b_per_w // _CHUNK
    base = wid * b_per_w

    def step(g, carry):
        off = base + g * _CHUNK
        pltpu.sync_copy(ids_hbm.at[pl.ds(off, _CHUNK)], idx_v)
        pltpu.async_copy(table_hbm.at[idx_v], rows_v, sem).wait()
        dst = (off // (2 * _TCBLK)) * _TCBLK + off % _TCBLK
        col = ((off // _TCBLK) % 2) * D
        pltpu.sync_copy(rows_v,
                        out_hbm.at[pl.ds(dst, _CHUNK), pl.ds(col, D)])
        return carry

    lax.fori_loop(0, n_chunks, step, 0)


@functools.cache
def _make_gather(n):
    mesh = plsc.VectorSubcoreMesh(core_axis_name="c", subcore_axis_name="s")
    return pl.kernel(
        _gather_body,
        mesh=mesh,
        out_type=jax.ShapeDtypeStruct((n // 2, 2 * D), jnp.float32),
        scratch_types=[
            pltpu.VMEM((_CHUNK,), jnp.int32),
            pltpu.VMEM((_CHUNK, D), jnp.float32),
            pltpu.SemaphoreType.DMA,
        ],
        compiler_params=pltpu.CompilerParams(use_tc_tiling_on_sc=False),
    )


def _proj_body(x_ref, w_ref, o_ref):
    blk = x_ref.shape[0]
    w = w_ref[...]
    dims = (((1,), (1,)), ((), ()))
    o_ref[0:blk, :] = lax.dot_general(
        x_ref[:, 0:D], w, dimension_numbers=dims,
        preferred_element_type=jnp.float32)
    o_ref[blk:2 * blk, :] = lax.dot_general(
        x_ref[:, D:2 * D], w, dimension_numbers=dims,
        preferred_element_type=jnp.float32)


def _project(x2, w):
    n2 = x2.shape[0]
    return pl.pallas_call(
        _proj_body,
        grid=(n2 // _TCBLK,),
        in_specs=[
            pl.BlockSpec((_TCBLK, 2 * D), lambda i: (i, 0)),
            pl.BlockSpec((M, D), lambda i: (0, 0)),
        ],
        out_specs=pl.BlockSpec((2 * _TCBLK, M), lambda i: (i, 0)),
        out_shape=jax.ShapeDtypeStruct((2 * n2, M), jnp.float32),
    )(x2, w)


def kernel(input_ids, low_rank_embed, projection_w):
    bsz, seq = input_ids.shape
    ids = input_ids.reshape(-1).astype(jnp.int32)
    n = ids.shape[0]
    rows2 = _make_gather(n)(low_rank_embed, ids)
    out = _project(rows2, projection_w)
    return out.reshape(bsz, seq, M)


# speedup vs baseline: 1.0106x; 1.0106x over previous
import functools

import jax
import jax.numpy as jnp
from jax import lax
from jax.experimental import pallas as pl
from jax.experimental.pallas import tpu as pltpu
from jax.experimental.pallas import tpu_sc as plsc

D = 64
M = 128

_NC = 2
_NS = 16
_NW = _NC * _NS

_CHUNK = 512
_TCBLK = 4096


def _gather_body(table_hbm, ids_hbm, out_hbm, idx_v, rows_v, sem):
    wid = lax.axis_index("s") * _NC + lax.axis_index("c")
    n = ids_hbm.shape[0]
    b_per_w = n // _NW
    n_chunks = b_per_w // _CHUNK
    base = wid * b_per_w

    def step(g, carry):
        off = base + g * _CHUNK
        pltpu.sync_copy(ids_hbm.at[pl.ds(off, _CHUNK)], idx_v)
        pltpu.async_copy(table_hbm.at[idx_v], rows_v, sem).wait()
        dst = (off // (2 * _TCBLK)) * _TCBLK + off % _TCBLK
        col = ((off // _TCBLK) % 2) * D
        pltpu.sync_copy(rows_v,
                        out_hbm.at[pl.ds(dst, _CHUNK), pl.ds(col, D)])
        return carry

    lax.fori_loop(0, n_chunks, step, 0)


@functools.cache
def _make_gather(n):
    mesh = plsc.VectorSubcoreMesh(core_axis_name="c", subcore_axis_name="s")
    return pl.kernel(
        _gather_body,
        mesh=mesh,
        out_type=jax.ShapeDtypeStruct((n // 2, 2 * D), jnp.float32),
        scratch_types=[
            pltpu.VMEM((_CHUNK,), jnp.int32),
            pltpu.VMEM((_CHUNK, D), jnp.float32),
            pltpu.SemaphoreType.DMA,
        ],
        compiler_params=pltpu.CompilerParams(use_tc_tiling_on_sc=False),
    )


def _proj_body(xa_ref, xb_ref, w_ref, o_ref):
    blk = xa_ref.shape[0]
    w = w_ref[...]
    dims = (((1,), (1,)), ((), ()))
    for h, x_ref in ((0, xa_ref), (1, xb_ref)):
        o_ref[(2 * h) * blk:(2 * h + 1) * blk, :] = lax.dot_general(
            x_ref[:, 0:D], w, dimension_numbers=dims,
            preferred_element_type=jnp.float32)
        o_ref[(2 * h + 1) * blk:(2 * h + 2) * blk, :] = lax.dot_general(
            x_ref[:, D:2 * D], w, dimension_numbers=dims,
            preferred_element_type=jnp.float32)


def _project(x2, w):
    n2 = x2.shape[0]
    return pl.pallas_call(
        _proj_body,
        grid=(n2 // (2 * _TCBLK),),
        in_specs=[
            pl.BlockSpec((_TCBLK, 2 * D), lambda i: (2 * i, 0)),
            pl.BlockSpec((_TCBLK, 2 * D), lambda i: (2 * i + 1, 0)),
            pl.BlockSpec((M, D), lambda i: (0, 0)),
        ],
        out_specs=pl.BlockSpec((4 * _TCBLK, M), lambda i: (i, 0)),
        out_shape=jax.ShapeDtypeStruct((2 * n2, M), jnp.float32),
    )(x2, x2, w)


def kernel(input_ids, low_rank_embed, projection_w):
    bsz, seq = input_ids.shape
    ids = input_ids.reshape(-1).astype(jnp.int32)
    n = ids.shape[0]
    rows2 = _make_gather(n)(low_rank_embed, ids)
    out = _project(rows2, projection_w)
    return out.reshape(bsz, seq, M)


# speedup vs baseline: 1.0498x; 1.0388x over previous
import functools

import jax
import jax.numpy as jnp
from jax import lax
from jax.experimental import pallas as pl
from jax.experimental.pallas import tpu as pltpu
from jax.experimental.pallas import tpu_sc as plsc

D = 64
M = 128

_NC = 2
_NS = 16
_NW = _NC * _NS

_CHUNK = 512
_TCBLK = 8192
_NSLICE = 5


def _gather_body(table_hbm, ids_hbm, out_hbm, idx_v, rows_v, sem):
    wid = lax.axis_index("s") * _NC + lax.axis_index("c")
    n = ids_hbm.shape[0]
    b_per_w = n // _NW
    n_chunks = b_per_w // _CHUNK
    base = wid * b_per_w

    def step(g, carry):
        off = base + g * _CHUNK
        pltpu.sync_copy(ids_hbm.at[pl.ds(off, _CHUNK)], idx_v)
        pltpu.async_copy(table_hbm.at[idx_v], rows_v, sem).wait()
        dst = (off // (2 * _TCBLK)) * _TCBLK + off % _TCBLK
        col = ((off // _TCBLK) % 2) * D
        pltpu.sync_copy(rows_v,
                        out_hbm.at[pl.ds(dst, _CHUNK), pl.ds(col, D)])
        return carry

    lax.fori_loop(0, n_chunks, step, 0)


@functools.cache
def _make_gather(n):
    mesh = plsc.VectorSubcoreMesh(core_axis_name="c", subcore_axis_name="s")
    return pl.kernel(
        _gather_body,
        mesh=mesh,
        out_type=jax.ShapeDtypeStruct((n // 2, 2 * D), jnp.float32),
        scratch_types=[
            pltpu.VMEM((_CHUNK,), jnp.int32),
            pltpu.VMEM((_CHUNK, D), jnp.float32),
            pltpu.SemaphoreType.DMA,
        ],
        compiler_params=pltpu.CompilerParams(use_tc_tiling_on_sc=False),
    )


def _proj_compute(x_ref, w_ref, o_ref):
    blk = x_ref.shape[0]
    w = w_ref[...]
    dims = (((1,), (1,)), ((), ()))
    o_ref[0:blk, :] = lax.dot_general(
        x_ref[:, 0:D], w, dimension_numbers=dims,
        preferred_element_type=jnp.float32)
    o_ref[blk:2 * blk, :] = lax.dot_general(
        x_ref[:, D:2 * D], w, dimension_numbers=dims,
        preferred_element_type=jnp.float32)


def _proj_first_body(x_ref, w_ref, o_ref):
    _proj_compute(x_ref, w_ref, o_ref)


def _proj_chain_body(acc_ref, x_ref, w_ref, o_ref):
    del acc_ref
    _proj_compute(x_ref, w_ref, o_ref)


def _project_slice(acc, x2, w, n_total, s):
    n2 = x2.shape[0]
    steps = n2 // _TCBLK
    off_blocks = s * steps
    out_shape = jax.ShapeDtypeStruct((n_total, M), jnp.float32)
    x_spec = pl.BlockSpec((_TCBLK, 2 * D), lambda i: (i, 0))
    w_spec = pl.BlockSpec((M, D), lambda i: (0, 0))
    o_spec = pl.BlockSpec((2 * _TCBLK, M), lambda i: (i + off_blocks, 0))
    if acc is None:
        return pl.pallas_call(
            _proj_first_body,
            grid=(steps,),
            in_specs=[x_spec, w_spec],
            out_specs=o_spec,
            out_shape=out_shape,
        )(x2, w)
    return pl.pallas_call(
        _proj_chain_body,
        grid=(steps,),
        in_specs=[pl.BlockSpec(memory_space=pl.ANY), x_spec, w_spec],
        out_specs=o_spec,
        out_shape=out_shape,
        input_output_aliases={0: 0},
    )(acc, x2, w)


def kernel(input_ids, low_rank_embed, projection_w):
    bsz, seq = input_ids.shape
    ids = input_ids.reshape(-1).astype(jnp.int32)
    n = ids.shape[0]
    n_s = n // _NSLICE
    gather = _make_gather(n_s)
    slices = [gather(low_rank_embed,
                     lax.slice(ids, (s * n_s,), ((s + 1) * n_s,)))
              for s in range(_NSLICE)]
    acc = None
    for s, x2 in enumerate(slices):
        acc = _project_slice(acc, x2, projection_w, n, s)
    return acc.reshape(bsz, seq, M)


# speedup vs baseline: 1.6747x; 1.5952x over previous
import functools

import jax
import jax.numpy as jnp
from jax import lax
from jax.experimental import pallas as pl
from jax.experimental.pallas import tpu as pltpu
from jax.experimental.pallas import tpu_sc as plsc

D = 64
M = 128

_NC = 2
_NS = 16
_NW = _NC * _NS

_CHUNK = 512
_PBLK = 16384


def _pt_main_body(vt_ref, w_ref, o_ref):
    o_ref[...] = lax.dot_general(
        vt_ref[...], w_ref[...],
        dimension_numbers=(((0,), (1,)), ((), ())),
        preferred_element_type=jnp.float32)


def _project_table(vt, w):
    v = vt.shape[1]
    return pl.pallas_call(
        _pt_main_body,
        grid=(pl.cdiv(v, _PBLK),),
        in_specs=[
            pl.BlockSpec((D, _PBLK), lambda i: (0, i)),
            pl.BlockSpec((M, D), lambda i: (0, 0)),
        ],
        out_specs=pl.BlockSpec((_PBLK, M), lambda i: (i, 0)),
        out_shape=jax.ShapeDtypeStruct((v, M), jnp.float32),
    )(vt, w)


def _gather_body(table_hbm, ids_hbm, out_hbm, idx_v, rows_v, sem):
    wid = lax.axis_index("s") * _NC + lax.axis_index("c")
    n = ids_hbm.shape[0]
    b_per_w = n // _NW
    n_chunks = b_per_w // _CHUNK
    base = wid * b_per_w

    def step(g, carry):
        off = base + g * _CHUNK
        pltpu.sync_copy(ids_hbm.at[pl.ds(off, _CHUNK)], idx_v)
        pltpu.async_copy(table_hbm.at[idx_v], rows_v, sem).wait()
        pltpu.sync_copy(rows_v, out_hbm.at[pl.ds(off, _CHUNK)])
        return carry

    lax.fori_loop(0, n_chunks, step, 0)


@functools.cache
def _make_gather(n, v):
    mesh = plsc.VectorSubcoreMesh(core_axis_name="c", subcore_axis_name="s")
    return pl.kernel(
        _gather_body,
        mesh=mesh,
        out_type=jax.ShapeDtypeStruct((n, M), jnp.float32),
        scratch_types=[
            pltpu.VMEM((_CHUNK,), jnp.int32),
            pltpu.VMEM((_CHUNK, M), jnp.float32),
            pltpu.SemaphoreType.DMA,
        ],
        compiler_params=pltpu.CompilerParams(use_tc_tiling_on_sc=False),
    )


def kernel(input_ids, low_rank_embed, projection_w):
    bsz, seq = input_ids.shape
    ids = input_ids.reshape(-1).astype(jnp.int32)
    n = ids.shape[0]
    v = low_rank_embed.shape[0]
    vt = low_rank_embed.T
    ptable = _project_table(vt, projection_w)
    out = _make_gather(n, v)(ptable, ids)
    return out.reshape(bsz, seq, M)


# speedup vs baseline: 1.7884x; 1.0679x over previous
import functools

import jax
import jax.numpy as jnp
from jax import lax
from jax.experimental import pallas as pl
from jax.experimental.pallas import tpu as pltpu
from jax.experimental.pallas import tpu_sc as plsc

D = 64
M = 128

_NC = 2
_NS = 16
_NW = _NC * _NS

_CHUNK = 320
_PBLK = 16384


def _pt_main_body(vt_ref, w_ref, o_ref):
    o_ref[...] = lax.dot_general(
        vt_ref[...], w_ref[...],
        dimension_numbers=(((0,), (1,)), ((), ())),
        preferred_element_type=jnp.float32)


def _project_table(vt, w):
    v = vt.shape[1]
    return pl.pallas_call(
        _pt_main_body,
        grid=(pl.cdiv(v, _PBLK),),
        in_specs=[
            pl.BlockSpec((D, _PBLK), lambda i: (0, i)),
            pl.BlockSpec((M, D), lambda i: (0, 0)),
        ],
        out_specs=pl.BlockSpec((_PBLK, M), lambda i: (i, 0)),
        out_shape=jax.ShapeDtypeStruct((v, M), jnp.float32),
    )(vt, w)


def _gather_body(table_hbm, ids_hbm, out_hbm, idx_v,
                 rows0, rows1, sg0, sg1, sw0, sw1):
    wid = lax.axis_index("s") * _NC + lax.axis_index("c")
    n = ids_hbm.shape[0]
    b_per_w = n // _NW
    n_chunks = b_per_w // _CHUNK
    base = wid * b_per_w
    rows = (rows0, rows1)
    sg = (sg0, sg1)
    sw = (sw0, sw1)

    def idx_at(g):
        return idx_v.at[pl.ds(g * _CHUNK, _CHUNK)]

    def out_at(g):
        return out_hbm.at[pl.ds(base + g * _CHUNK, _CHUNK)]

    pltpu.sync_copy(ids_hbm.at[pl.ds(base, b_per_w)], idx_v)
    for b in range(2):
        pltpu.async_copy(table_hbm.at[idx_at(b)], rows[b], sg[b])

    def pair(p, carry):
        g0 = p * 2
        for b in range(2):
            g = g0 + b
            pltpu.make_async_copy(table_hbm.at[idx_at(g)], rows[b],
                                  sg[b]).wait()
            pltpu.async_copy(rows[b], out_at(g), sw[b])

            @pl.when(g + 2 < n_chunks)
            def _():
                pltpu.make_async_copy(rows[b], out_at(g), sw[b]).wait()
                pltpu.async_copy(table_hbm.at[idx_at(g + 2)], rows[b], sg[b])
        return carry

    lax.fori_loop(0, n_chunks // 2, pair, 0)
    for b in range(2):
        g = n_chunks - 2 + b
        pltpu.make_async_copy(rows[b], out_at(g), sw[b]).wait()


@functools.cache
def _make_gather(n, v):
    mesh = plsc.VectorSubcoreMesh(core_axis_name="c", subcore_axis_name="s")
    b_per_w = n // _NW
    return pl.kernel(
        _gather_body,
        mesh=mesh,
        out_type=jax.ShapeDtypeStruct((n, M), jnp.float32),
        scratch_types=[
            pltpu.VMEM((b_per_w,), jnp.int32),
            pltpu.VMEM((_CHUNK, M), jnp.float32),
            pltpu.VMEM((_CHUNK, M), jnp.float32),
            pltpu.SemaphoreType.DMA,
            pltpu.SemaphoreType.DMA,
            pltpu.SemaphoreType.DMA,
            pltpu.SemaphoreType.DMA,
        ],
        compiler_params=pltpu.CompilerParams(use_tc_tiling_on_sc=False),
    )


def kernel(input_ids, low_rank_embed, projection_w):
    bsz, seq = input_ids.shape
    ids = input_ids.reshape(-1).astype(jnp.int32)
    n = ids.shape[0]
    v = low_rank_embed.shape[0]
    vt = low_rank_embed.T
    ptable = _project_table(vt, projection_w)
    out = _make_gather(n, v)(ptable, ids)
    return out.reshape(bsz, seq, M)


# speedup vs baseline: 1.8131x; 1.0138x over previous
import functools

import jax
import jax.numpy as jnp
from jax import lax
from jax.experimental import pallas as pl
from jax.experimental.pallas import tpu as pltpu
from jax.experimental.pallas import tpu_sc as plsc

D = 64
M = 128

_NC = 2
_NS = 16
_NW = _NC * _NS

_CHUNK = 160
_PBLK = 32768


def _pt_main_body(vt_ref, w_ref, o_ref):
    o_ref[...] = lax.dot_general(
        vt_ref[...], w_ref[...],
        dimension_numbers=(((0,), (1,)), ((), ())),
        preferred_element_type=jnp.float32)


def _project_table(vt, w):
    v = vt.shape[1]
    return pl.pallas_call(
        _pt_main_body,
        grid=(pl.cdiv(v, _PBLK),),
        in_specs=[
            pl.BlockSpec((D, _PBLK), lambda i: (0, i)),
            pl.BlockSpec((M, D), lambda i: (0, 0)),
        ],
        out_specs=pl.BlockSpec((_PBLK, M), lambda i: (i, 0)),
        out_shape=jax.ShapeDtypeStruct((v, M), jnp.float32),
    )(vt, w)


def _gather_body(table_hbm, ids_hbm, out_hbm, idx_v,
                 rows0, rows1, rows2, rows3,
                 sg0, sg1, sg2, sg3, sw0, sw1, sw2, sw3):
    wid = lax.axis_index("s") * _NC + lax.axis_index("c")
    n = ids_hbm.shape[0]
    b_per_w = n // _NW
    n_chunks = b_per_w // _CHUNK
    base = wid * b_per_w
    rows = (rows0, rows1, rows2, rows3)
    sg = (sg0, sg1, sg2, sg3)
    sw = (sw0, sw1, sw2, sw3)
    nbuf = 4

    def idx_at(g):
        return idx_v.at[pl.ds(g * _CHUNK, _CHUNK)]

    def out_at(g):
        return out_hbm.at[pl.ds(base + g * _CHUNK, _CHUNK)]

    pltpu.sync_copy(ids_hbm.at[pl.ds(base, b_per_w)], idx_v)
    for b in range(nbuf):
        pltpu.async_copy(table_hbm.at[idx_at(b)], rows[b], sg[b])

    def ring(p, carry):
        g0 = p * nbuf
        for b in range(nbuf):
            g = g0 + b
            pltpu.make_async_copy(table_hbm.at[idx_at(g)], rows[b],
                                  sg[b]).wait()
            pltpu.async_copy(rows[b], out_at(g), sw[b])

            @pl.when(g + nbuf < n_chunks)
            def _():
                pltpu.make_async_copy(rows[b], out_at(g), sw[b]).wait()
                pltpu.async_copy(table_hbm.at[idx_at(g + nbuf)], rows[b],
                                 sg[b])
        return carry

    lax.fori_loop(0, n_chunks // nbuf, ring, 0)
    for b in range(nbuf):
        g = n_chunks - nbuf + b
        pltpu.make_async_copy(rows[b], out_at(g), sw[b]).wait()


@functools.cache
def _make_gather(n, v):
    mesh = plsc.VectorSubcoreMesh(core_axis_name="c", subcore_axis_name="s")
    b_per_w = n // _NW
    return pl.kernel(
        _gather_body,
        mesh=mesh,
        out_type=jax.ShapeDtypeStruct((n, M), jnp.float32),
        scratch_types=(
            [pltpu.VMEM((b_per_w,), jnp.int32)]
            + [pltpu.VMEM((_CHUNK, M), jnp.float32)] * 4
            + [pltpu.SemaphoreType.DMA] * 8
        ),
        compiler_params=pltpu.CompilerParams(use_tc_tiling_on_sc=False),
    )


def kernel(input_ids, low_rank_embed, projection_w):
    bsz, seq = input_ids.shape
    ids = input_ids.reshape(-1).astype(jnp.int32)
    n = ids.shape[0]
    v = low_rank_embed.shape[0]
    vt = low_rank_embed.T
    ptable = _project_table(vt, projection_w)
    out = _make_gather(n, v)(ptable, ids)
    return out.reshape(bsz, seq, M)
